# baseline (speedup 1.0000x reference)
import functools

import jax
import jax.numpy as jnp
from jax import lax
from jax.experimental import pallas as pl
from jax.experimental.pallas import tpu as pltpu
from jax.experimental.pallas import tpu_sc as plsc

_D = 32
_R = 16384
_C = 50
_SKEW = 137

_info = plsc.get_sparse_core_info()
_NC, _NS = _info.num_cores, _info.num_subcores
_NW = _NC * _NS
_I_PER_W = _R // _NW
_ICW = _I_PER_W // 128
_UH = _ICW // 2
_NU = _C * _UH


@functools.partial(
    pl.kernel,
    out_type=jax.ShapeDtypeStruct((_C, _D // 8, _R // 128, 8, 128), jnp.float32),
    mesh=plsc.VectorSubcoreMesh(core_axis_name="c", subcore_axis_name="s"),
    scratch_types=[
        pltpu.VMEM((_I_PER_W, _C), jnp.int32),
        pltpu.VMEM((2, 256), jnp.int32),
        pltpu.VMEM((2, 256, _D), jnp.float32),
        pltpu.VMEM((2, 2, _D, _SKEW), jnp.float32),
        pltpu.SemaphoreType.DMA,
        pltpu.SemaphoreType.DMA,
        pltpu.SemaphoreType.DMA,
        pltpu.SemaphoreType.DMA,
    ],
    compiler_params=pltpu.CompilerParams(
        use_tc_tiling_on_sc=False, needs_layout_passes=False
    ),
)
def _emb_gather(x_hbm, table_hbm, out_hbm, xv, fidx, rows, tbuf, g0, g1, s0, s1):
    gsem = (g0, g1)
    ssem = (s0, s1)
    wid = lax.axis_index("s") * _NC + lax.axis_index("c")
    i0 = wid * _I_PER_W

    pltpu.sync_copy(x_hbm.at[pl.ds(i0, _I_PER_W), :], xv)

    lane = lax.iota(jnp.int32, 16)

    def build_and_fire(u, b):
        j = u // _UH
        ip = u % _UH
        cvec = lane * 0 + j
        for k in range(16):
            rvec = lane + (ip * 256 + k * 16)
            fidx[b, pl.ds(k * 16, 16)] = plsc.load_gather(xv, [rvec, cvec])
        pltpu.async_copy(table_hbm.at[fidx.at[b]], rows.at[b], gsem[b])

    def wait_gather(b):
        pltpu.make_async_copy(
            table_hbm.at[fidx.at[b]], rows.at[b], gsem[b]
        ).wait()

    def store_tiles(u, b):
        j = u // _UH
        ic0 = wid * _ICW + (u % _UH) * 2
        for h in range(2):
            for dr in range(_D // 8):
                pltpu.async_copy(
                    tbuf.at[b, h, pl.ds(dr * 8, 8), pl.ds(0, 128)],
                    out_hbm.at[j, dr, ic0 + h],
                    ssem[b],
                )

    def wait_tiles(b):
        for _ in range(2 * (_D // 8)):
            pltpu.make_async_copy(
                tbuf.at[b, 0, pl.ds(0, 8), pl.ds(0, 128)],
                out_hbm.at[0, 0, 0],
                ssem[b],
            ).wait()

    def transpose(b):
        for h in range(2):
            for il in range(128):
                ilvec = lane * 0 + il
                for k in range(2):
                    dvec = lane + k * 16
                    plsc.store_scatter(
                        tbuf.at[b, h],
                        [dvec, ilvec],
                        rows[b, h * 128 + il, pl.ds(k * 16, 16)],
                    )

    build_and_fire(0, 0)
    nt = _NU // 2

    def body(t, carry):
        u0 = 2 * t
        build_and_fire(u0 + 1, 1)
        wait_gather(0)

        @pl.when(t >= 1)
        def _():
            wait_tiles(0)

        transpose(0)
        store_tiles(u0, 0)

        @pl.when(t < nt - 1)
        def _():
            build_and_fire(u0 + 2, 0)

        wait_gather(1)

        @pl.when(t >= 1)
        def _():
            wait_tiles(1)

        transpose(1)
        store_tiles(u0 + 1, 1)
        return carry

    lax.fori_loop(0, nt, body, 0)

    wait_tiles(0)
    wait_tiles(1)


def kernel(x, table):
    out5 = _emb_gather(x.astype(jnp.int32), table)
    return out5.transpose(2, 4, 0, 1, 3).reshape(_R, _C, _D)


# speedup vs baseline: 1.0078x; 1.0078x over previous
import functools

import jax
import jax.numpy as jnp
from jax import lax
from jax.experimental import pallas as pl
from jax.experimental.pallas import tpu as pltpu
from jax.experimental.pallas import tpu_sc as plsc

_D = 32
_R = 16384
_C = 50
_SKEW = 137
_NB = 4

_info = plsc.get_sparse_core_info()
_NC, _NS = _info.num_cores, _info.num_subcores
_NW = _NC * _NS
_I_PER_W = _R // _NW
_ICW = _I_PER_W // 128
_NU = _C * _ICW


@functools.partial(
    pl.kernel,
    out_type=jax.ShapeDtypeStruct((_C, _D // 8, _R // 128, 8, 128), jnp.float32),
    mesh=plsc.VectorSubcoreMesh(core_axis_name="c", subcore_axis_name="s"),
    scratch_types=[
        pltpu.VMEM((_I_PER_W, _C), jnp.int32),
        pltpu.VMEM((_NB, 128), jnp.int32),
        pltpu.VMEM((_NB, 128, _D), jnp.float32),
        pltpu.VMEM((_NB, _D, _SKEW), jnp.float32),
    ]
    + [pltpu.SemaphoreType.DMA] * (2 * _NB),
    compiler_params=pltpu.CompilerParams(
        use_tc_tiling_on_sc=False, needs_layout_passes=False
    ),
)
def _emb_gather(x_hbm, table_hbm, out_hbm, xv, fidx, rows, tbuf, *sems):
    gsem = sems[:_NB]
    ssem = sems[_NB:]
    wid = lax.axis_index("s") * _NC + lax.axis_index("c")
    i0 = wid * _I_PER_W

    pltpu.sync_copy(x_hbm.at[pl.ds(i0, _I_PER_W), :], xv)

    lane = lax.iota(jnp.int32, 16)

    def build_and_fire(u, b):
        j = u // _ICW
        icl = u % _ICW
        cvec = lane * 0 + j
        for k in range(8):
            rvec = lane + (icl * 128 + k * 16)
            fidx[b, pl.ds(k * 16, 16)] = plsc.load_gather(xv, [rvec, cvec])
        pltpu.async_copy(table_hbm.at[fidx.at[b]], rows.at[b], gsem[b])

    def wait_gather(b):
        pltpu.make_async_copy(
            table_hbm.at[fidx.at[b]], rows.at[b], gsem[b]
        ).wait()

    def store_tiles(u, b):
        j = u // _ICW
        ic = wid * _ICW + u % _ICW
        for dr in range(_D // 8):
            pltpu.async_copy(
                tbuf.at[b, pl.ds(dr * 8, 8), pl.ds(0, 128)],
                out_hbm.at[j, dr, ic],
                ssem[b],
            )

    def wait_tiles(b):
        for _ in range(_D // 8):
            pltpu.make_async_copy(
                tbuf.at[b, pl.ds(0, 8), pl.ds(0, 128)],
                out_hbm.at[0, 0, 0],
                ssem[b],
            ).wait()

    def transpose(b):
        for il in range(128):
            ilvec = lane * 0 + il
            for k in range(2):
                dvec = lane + k * 16
                plsc.store_scatter(
                    tbuf.at[b], [dvec, ilvec], rows[b, il, pl.ds(k * 16, 16)]
                )

    for b in range(_NB - 1):
        build_and_fire(b, b)
    nt = _NU // _NB

    def body(t, carry):
        u0 = _NB * t
        for q in range(_NB):
            u = u0 + q
            wait_gather(q)

            @pl.when(t >= 1)
            def _():
                wait_tiles(q)

            transpose(q)
            store_tiles(u, q)

            @pl.when(u + _NB - 1 < _NU)
            def _():
                build_and_fire(u + _NB - 1, (q + _NB - 1) % _NB)

        return carry

    lax.fori_loop(0, nt, body, 0)

    for b in range(_NB):
        wait_tiles(b)


def kernel(x, table):
    out5 = _emb_gather(x.astype(jnp.int32), table)
    return out5.transpose(2, 4, 0, 1, 3).reshape(_R, _C, _D)


# speedup vs baseline: 1.0572x; 1.0490x over previous
import functools

import jax
import jax.numpy as jnp
from jax import lax
from jax.experimental import pallas as pl
from jax.experimental.pallas import tpu as pltpu
from jax.experimental.pallas import tpu_sc as plsc

_D = 32
_R = 16384
_C = 50
_SKEW = 137

_info = plsc.get_sparse_core_info()
_NC, _NS = _info.num_cores, _info.num_subcores
_NW = _NC * _NS
_I_PER_W = _R // _NW
_ICW = _I_PER_W // 128
_NU = _C * _ICW


@functools.partial(
    pl.kernel,
    out_type=jax.ShapeDtypeStruct((_C, _D // 8, _R // 128, 8, 128), jnp.float32),
    mesh=plsc.VectorSubcoreMesh(core_axis_name="c", subcore_axis_name="s"),
    scratch_types=[
        pltpu.VMEM((_I_PER_W, _C), jnp.int32),
        pltpu.VMEM((_C, _I_PER_W + 11), jnp.int32),
        pltpu.VMEM((2, 128, _D), jnp.float32),
        pltpu.VMEM((2, _D, _SKEW), jnp.float32),
        pltpu.SemaphoreType.DMA,
        pltpu.SemaphoreType.DMA,
        pltpu.SemaphoreType.DMA,
        pltpu.SemaphoreType.DMA,
    ],
    compiler_params=pltpu.CompilerParams(
        use_tc_tiling_on_sc=False, needs_layout_passes=False
    ),
)
def _emb_gather(x_hbm, table_hbm, out_hbm, xv, xvt, rows, tbuf, g0, g1, s0, s1):
    gsem = (g0, g1)
    ssem = (s0, s1)
    wid = lax.axis_index("s") * _NC + lax.axis_index("c")
    i0 = wid * _I_PER_W

    pltpu.sync_copy(x_hbm.at[pl.ds(i0, _I_PER_W), :], xv)

    lane = lax.iota(jnp.int32, 16)

    def xpose(r, carry):
        rvec = lane * 0 + r
        for k in (0, 16, 32, 34):
            v = plsc.load_gather(xv, [rvec, lane + k])
            plsc.store_scatter(xvt, [lane + k, rvec], v)
        return carry

    lax.fori_loop(0, _I_PER_W, xpose, 0)

    def _idx(u):
        j = u // _ICW
        icl = u % _ICW
        return xvt.at[j, pl.ds(icl * 128, 128)]

    def build_and_fire(u, b):
        pltpu.async_copy(table_hbm.at[_idx(u)], rows.at[b], gsem[b])

    def wait_gather(b):
        pltpu.make_async_copy(
            table_hbm.at[_idx(0)], rows.at[b], gsem[b]
        ).wait()

    def store_tiles(u, b):
        j = u // _ICW
        ic = wid * _ICW + u % _ICW
        for dr in range(_D // 8):
            pltpu.async_copy(
                tbuf.at[b, pl.ds(dr * 8, 8), pl.ds(0, 128)],
                out_hbm.at[j, dr, ic],
                ssem[b],
            )

    def wait_tiles(b):
        for _ in range(_D // 8):
            pltpu.make_async_copy(
                tbuf.at[b, pl.ds(0, 8), pl.ds(0, 128)],
                out_hbm.at[0, 0, 0],
                ssem[b],
            ).wait()

    def transpose(b):
        for il in range(128):
            ilvec = lane * 0 + il
            for k in range(2):
                dvec = lane + k * 16
                plsc.store_scatter(
                    tbuf.at[b], [dvec, ilvec], rows[b, il, pl.ds(k * 16, 16)]
                )

    build_and_fire(0, 0)
    nt = _NU // 2

    def body(t, carry):
        u0 = 2 * t
        build_and_fire(u0 + 1, 1)
        wait_gather(0)

        @pl.when(t >= 1)
        def _():
            wait_tiles(0)

        transpose(0)
        store_tiles(u0, 0)

        @pl.when(t < nt - 1)
        def _():
            build_and_fire(u0 + 2, 0)

        wait_gather(1)

        @pl.when(t >= 1)
        def _():
            wait_tiles(1)

        transpose(1)
        store_tiles(u0 + 1, 1)
        return carry

    lax.fori_loop(0, nt, body, 0)

    wait_tiles(0)
    wait_tiles(1)


def kernel(x, table):
    out5 = _emb_gather(x.astype(jnp.int32), table)
    return out5.transpose(2, 4, 0, 1, 3).reshape(_R, _C, _D)
